# baseline (speedup 1.0000x reference)
import functools

import jax
import jax.numpy as jnp
from jax.experimental import pallas as pl

N = 10000
E = 160000
NUM_SPECIES = 10
FDIM = 128
H = 76
NB = 8
SH = 16
NL = 4
Q = 8
QL = 3
MH = 64
CUTOFF = 5.0
AVG = 16.0

HP = 80
BE = 2000


def _edge_m_body(radial_ref, attr_ref, w1_ref, w2_ref, w3_ref, wep_ref, out_ref):
    r = radial_ref[...]
    a = attr_ref[...]
    h = r @ w1_ref[...]
    h = h * jax.nn.sigmoid(h)
    h = h @ w2_ref[...]
    h = h * jax.nn.sigmoid(h)
    mix = h @ w3_ref[...]
    ep = a @ wep_ref[...]
    col = jax.lax.broadcasted_iota(jnp.int32, mix.shape, 1)
    scale = jnp.where(col < HP, 1.0, ep)
    out_ref[...] = mix * scale


def _edge_m(radial, attr, W1, W2, W3, W_edge):
    W3p = jnp.zeros((MH, 2 * HP), jnp.float32)
    W3p = W3p.at[:, :H].set(W3[:, :H]).at[:, HP:HP + H].set(W3[:, H:])
    Wep = jnp.zeros((SH, 2 * HP), jnp.float32)
    Wep = Wep.at[:, HP:HP + H].set(W_edge)
    grid = (E // BE,)
    return pl.pallas_call(
        _edge_m_body,
        grid=grid,
        in_specs=[
            pl.BlockSpec((BE, NB), lambda i: (i, 0)),
            pl.BlockSpec((BE, SH), lambda i: (i, 0)),
            pl.BlockSpec((NB, MH), lambda i: (0, 0)),
            pl.BlockSpec((MH, MH), lambda i: (0, 0)),
            pl.BlockSpec((MH, 2 * HP), lambda i: (0, 0)),
            pl.BlockSpec((SH, 2 * HP), lambda i: (0, 0)),
        ],
        out_specs=pl.BlockSpec((BE, 2 * HP), lambda i: (i, 0)),
        out_shape=jax.ShapeDtypeStruct((E, 2 * HP), jnp.float32),
    )(radial, attr, W1, W2, W3p, Wep)


def _bessel(x, n):
    x = jnp.clip(x, 1e-6, None)
    k = jnp.arange(1, n + 1, dtype=jnp.float32)
    return jnp.sqrt(2.0) * jnp.sin(k[None, :] * jnp.pi * x[:, None]) / x[:, None]


def _poly_envelope(x):
    p = 2.0
    y = (1.0 - (p + 1.0) * (p + 2.0) / 2.0 * x ** p
         + p * (p + 2.0) * x ** (p + 1.0)
         - p * (p + 1.0) / 2.0 * x ** (p + 2.0))
    return jnp.where(x < 1.0, y, 0.0)


def _sph_harm16(u):
    x, y, z = u[:, 0], u[:, 1], u[:, 2]
    l0 = jnp.ones_like(x)[:, None]
    c1 = jnp.sqrt(3.0)
    l1 = jnp.stack([c1 * x, c1 * y, c1 * z], axis=1)
    c2 = jnp.sqrt(15.0)
    l2 = jnp.stack([c2 * x * y, c2 * y * z, jnp.sqrt(5.0) / 2.0 * (3.0 * z * z - 1.0),
                    c2 * x * z, c2 / 2.0 * (x * x - y * y)], axis=1)
    c3 = jnp.sqrt(7.0)
    l3 = jnp.stack([x * (x * x - 3.0 * y * y), y * (3.0 * x * x - y * y),
                    z * (x * x - y * y), x * y * z, x * (5.0 * z * z - 1.0),
                    y * (5.0 * z * z - 1.0), z * (5.0 * z * z - 3.0)], axis=1) * c3
    return jnp.concatenate([l0, l1, l2, l3], axis=1)


def _qml_cx(feats, wq):
    for l in range(QL):
        feats = jnp.cos(feats * wq[l]) * jnp.sin(jnp.roll(feats, 1, axis=-1) + wq[l])
    return feats


def kernel(Rij, species, senders, receivers, n_node, params):
    R = Rij / CUTOFF
    lengths = jnp.linalg.norm(R, axis=1)
    radial = jnp.where((lengths == 0.0)[:, None], 0.0,
                       _bessel(lengths, NB) * _poly_envelope(lengths)[:, None])
    u = R / jnp.where(lengths == 0.0, 1.0, lengths)[:, None]
    edges_attr = _sph_harm16(u)

    node_feats = params['embed'][species]
    x_node = node_feats @ params['W_xlin']
    oh = jax.nn.one_hot(species, NUM_SPECIES, dtype=jnp.float32)

    outputs = []
    for l in range(NL):
        p = params['layers'][l]
        m = _edge_m(radial, edges_attr, p['W1'], p['W2'], p['W3'], p['W_edge'])

        proj = jnp.einsum('nd,sdh->nsh', node_feats, p['W_skip'])
        skip = jnp.einsum('nsh,ns->nh', proj, oh)
        h = node_feats @ p['W_up']

        hs = h[senders]
        msg = jnp.concatenate([hs * m[:, :H], hs * m[:, HP:HP + H]], axis=1)
        agg = jnp.zeros((N, 2 * H), jnp.float32).at[receivers].add(msg)
        agg = agg / jnp.sqrt(AVG)

        h2 = (agg @ p['W_down']) / jnp.sqrt(AVG)
        h2 = h2 * (x_node @ p['W_x'])
        h2 = h2 @ p['W_lin2']
        nf = h2 + skip
        feats = nf @ p['W_q']
        feats = _qml_cx(feats, p['wq'])
        out = feats @ p['W_out'] + p['b_out']
        outputs.append(out[:, 0])
        node_feats = nf

    node_energy = jnp.stack(outputs, axis=1).sum(axis=-1)
    seg = jnp.repeat(jnp.arange(n_node.shape[0]), n_node, total_repeat_length=N)
    graph_energy = jax.ops.segment_sum(node_energy, seg, num_segments=n_node.shape[0])
    node_logvar = jnp.zeros((N,), jnp.float32)
    graph_var = jax.ops.segment_sum(jnp.exp(node_logvar), seg,
                                    num_segments=n_node.shape[0]) / n_node
    return (graph_energy.reshape(-1), graph_var.reshape(-1))


# speedup vs baseline: 2.3412x; 2.3412x over previous
import functools

import jax
import jax.numpy as jnp
from jax import lax
from jax.experimental import pallas as pl
from jax.experimental.pallas import tpu as pltpu
from jax.experimental.pallas import tpu_sc as plsc

N = 10000
E = 160000
NUM_SPECIES = 10
FDIM = 128
H = 76
NB = 8
SH = 16
NL = 4
Q = 8
QL = 3
MH = 64
CUTOFF = 5.0
AVG = 16.0

HP = 80
MW = 2 * HP
BE = 2000

NC = 2
NS = 16
NW = NC * NS
CHUNK = 128
NCHUNK = E // CHUNK
TILE_ROWS = 624


def _edge_m_body(radial_ref, attr_ref, w1_ref, w2_ref, w3_ref, wep_ref, out_ref):
    r = radial_ref[...]
    a = attr_ref[...]
    h = r @ w1_ref[...]
    h = h * jax.nn.sigmoid(h)
    h = h @ w2_ref[...]
    h = h * jax.nn.sigmoid(h)
    mix = h @ w3_ref[...]
    ep = a @ wep_ref[...]
    col = jax.lax.broadcasted_iota(jnp.int32, mix.shape, 1)
    scale = jnp.where(col < HP, 1.0, ep)
    out_ref[...] = mix * scale


def _edge_m(radial, attr, W1, W2, W3, W_edge):
    W3p = jnp.zeros((MH, MW), jnp.float32)
    W3p = W3p.at[:, :H].set(W3[:, :H]).at[:, HP:HP + H].set(W3[:, H:])
    Wep = jnp.zeros((SH, MW), jnp.float32)
    Wep = Wep.at[:, HP:HP + H].set(W_edge)
    grid = (E // BE,)
    return pl.pallas_call(
        _edge_m_body,
        grid=grid,
        in_specs=[
            pl.BlockSpec((BE, NB), lambda i: (i, 0)),
            pl.BlockSpec((BE, SH), lambda i: (i, 0)),
            pl.BlockSpec((NB, MH), lambda i: (0, 0)),
            pl.BlockSpec((MH, MH), lambda i: (0, 0)),
            pl.BlockSpec((MH, MW), lambda i: (0, 0)),
            pl.BlockSpec((SH, MW), lambda i: (0, 0)),
        ],
        out_specs=pl.BlockSpec((BE, MW), lambda i: (i, 0)),
        out_shape=jax.ShapeDtypeStruct((E, MW), jnp.float32),
    )(radial, attr, W1, W2, W3p, Wep)


def _sc_agg_body(h_hbm, m_hbm, snd_hbm, rcv_hbm, out_hbm,
                 idx_s, idx_r, rows, mbuf, agg_sh, sem):
    cid = lax.axis_index("c")
    sid = lax.axis_index("s")
    wid = sid * NC + cid

    def _zero_mbuf(k, _):
        for c in range(MW // 16):
            mbuf[k, pl.ds(16 * c, 16)] = jnp.zeros((16,), jnp.float32)
        return 0
    lax.fori_loop(0, CHUNK, _zero_mbuf, 0)
    start = sid * TILE_ROWS
    for t in range(4):
        pltpu.sync_copy(mbuf, agg_sh.at[pl.ds(start + t * CHUNK, CHUNK)])
    pltpu.sync_copy(mbuf.at[pl.ds(0, 112)], agg_sh.at[pl.ds(start + 512, 112)])

    @pl.when(sid == NS - 1)
    def _zero_tail():
        pltpu.sync_copy(mbuf.at[pl.ds(0, 16)], agg_sh.at[pl.ds(N - 16, 16)])
    plsc.subcore_barrier()

    def _chunk(j, _):
        chunk_id = wid + NW * j

        @pl.when(chunk_id < NCHUNK)
        def _():
            off = chunk_id * CHUNK
            pltpu.sync_copy(snd_hbm.at[pl.ds(off, CHUNK)], idx_s)
            pltpu.sync_copy(rcv_hbm.at[pl.ds(off, CHUNK)], idx_r)
            pltpu.async_copy(h_hbm.at[idx_s], rows, sem).wait()
            pltpu.sync_copy(m_hbm.at[pl.ds(off, CHUNK)], mbuf)

            def _edge(k, _):
                h0 = rows[k, pl.ds(0, 16)]
                h1 = rows[k, pl.ds(16, 16)]
                h2 = rows[k, pl.ds(32, 16)]
                h3 = rows[k, pl.ds(48, 16)]
                h4 = rows[k, pl.ds(64, 16)]
                hv = (h0, h1, h2, h3, h4)
                for c in range(MW // 16):
                    mbuf[k, pl.ds(16 * c, 16)] = mbuf[k, pl.ds(16 * c, 16)] * hv[c % 5]
                return 0
            lax.fori_loop(0, CHUNK, _edge, 0)

            pltpu.sync_copy(mbuf, agg_sh.at[idx_r], add=True)
        return 0
    lax.fori_loop(0, (NCHUNK + NW - 1) // NW, _chunk, 0)

    plsc.subcore_barrier()
    pltpu.sync_copy(agg_sh.at[pl.ds(start, TILE_ROWS)],
                    out_hbm.at[cid, pl.ds(start, TILE_ROWS)])

    @pl.when(sid == NS - 1)
    def _pub_tail():
        pltpu.sync_copy(agg_sh.at[pl.ds(N - 16, 16)],
                        out_hbm.at[cid, pl.ds(N - 16, 16)])


@functools.partial(jax.jit, static_argnames=())
def _sc_agg(h_pad, m, senders, receivers):
    mesh = plsc.VectorSubcoreMesh(core_axis_name="c", subcore_axis_name="s")
    f = pl.kernel(
        _sc_agg_body,
        out_type=jax.ShapeDtypeStruct((NC, N, MW), jnp.float32),
        mesh=mesh,
        scratch_types=[
            pltpu.VMEM((CHUNK,), jnp.int32),
            pltpu.VMEM((CHUNK,), jnp.int32),
            pltpu.VMEM((CHUNK, HP), jnp.float32),
            pltpu.VMEM((CHUNK, MW), jnp.float32),
            pltpu.VMEM_SHARED((N, MW), jnp.float32),
            pltpu.SemaphoreType.DMA,
        ],
        compiler_params=pltpu.CompilerParams(use_tc_tiling_on_sc=False),
    )
    return f(h_pad, m, senders, receivers)


def _bessel(x, n):
    x = jnp.clip(x, 1e-6, None)
    k = jnp.arange(1, n + 1, dtype=jnp.float32)
    return jnp.sqrt(2.0) * jnp.sin(k[None, :] * jnp.pi * x[:, None]) / x[:, None]


def _poly_envelope(x):
    p = 2.0
    y = (1.0 - (p + 1.0) * (p + 2.0) / 2.0 * x ** p
         + p * (p + 2.0) * x ** (p + 1.0)
         - p * (p + 1.0) / 2.0 * x ** (p + 2.0))
    return jnp.where(x < 1.0, y, 0.0)


def _sph_harm16(u):
    x, y, z = u[:, 0], u[:, 1], u[:, 2]
    l0 = jnp.ones_like(x)[:, None]
    c1 = jnp.sqrt(3.0)
    l1 = jnp.stack([c1 * x, c1 * y, c1 * z], axis=1)
    c2 = jnp.sqrt(15.0)
    l2 = jnp.stack([c2 * x * y, c2 * y * z, jnp.sqrt(5.0) / 2.0 * (3.0 * z * z - 1.0),
                    c2 * x * z, c2 / 2.0 * (x * x - y * y)], axis=1)
    c3 = jnp.sqrt(7.0)
    l3 = jnp.stack([x * (x * x - 3.0 * y * y), y * (3.0 * x * x - y * y),
                    z * (x * x - y * y), x * y * z, x * (5.0 * z * z - 1.0),
                    y * (5.0 * z * z - 1.0), z * (5.0 * z * z - 3.0)], axis=1) * c3
    return jnp.concatenate([l0, l1, l2, l3], axis=1)


def _qml_cx(feats, wq):
    for l in range(QL):
        feats = jnp.cos(feats * wq[l]) * jnp.sin(jnp.roll(feats, 1, axis=-1) + wq[l])
    return feats


def kernel(Rij, species, senders, receivers, n_node, params):
    R = Rij / CUTOFF
    lengths = jnp.linalg.norm(R, axis=1)
    radial = jnp.where((lengths == 0.0)[:, None], 0.0,
                       _bessel(lengths, NB) * _poly_envelope(lengths)[:, None])
    u = R / jnp.where(lengths == 0.0, 1.0, lengths)[:, None]
    edges_attr = _sph_harm16(u)

    node_feats = params['embed'][species]
    x_node = node_feats @ params['W_xlin']
    oh = jax.nn.one_hot(species, NUM_SPECIES, dtype=jnp.float32)

    outputs = []
    for l in range(NL):
        p = params['layers'][l]
        m = _edge_m(radial, edges_attr, p['W1'], p['W2'], p['W3'], p['W_edge'])

        proj = jnp.einsum('nd,sdh->nsh', node_feats, p['W_skip'])
        skip = jnp.einsum('nsh,ns->nh', proj, oh)
        h = node_feats @ p['W_up']
        h_pad = jnp.pad(h, ((0, 0), (0, HP - H)))

        parts = _sc_agg(h_pad, m, senders, receivers)
        aggp = parts[0] + parts[1]
        agg = jnp.concatenate([aggp[:, :H], aggp[:, HP:HP + H]], axis=1)
        agg = agg / jnp.sqrt(AVG)

        h2 = (agg @ p['W_down']) / jnp.sqrt(AVG)
        h2 = h2 * (x_node @ p['W_x'])
        h2 = h2 @ p['W_lin2']
        nf = h2 + skip
        feats = nf @ p['W_q']
        feats = _qml_cx(feats, p['wq'])
        out = feats @ p['W_out'] + p['b_out']
        outputs.append(out[:, 0])
        node_feats = nf

    node_energy = jnp.stack(outputs, axis=1).sum(axis=-1)
    seg = jnp.repeat(jnp.arange(n_node.shape[0]), n_node, total_repeat_length=N)
    graph_energy = jax.ops.segment_sum(node_energy, seg, num_segments=n_node.shape[0])
    node_logvar = jnp.zeros((N,), jnp.float32)
    graph_var = jax.ops.segment_sum(jnp.exp(node_logvar), seg,
                                    num_segments=n_node.shape[0]) / n_node
    return (graph_energy.reshape(-1), graph_var.reshape(-1))
